# baseline (speedup 1.0000x reference)
import functools

import jax
import jax.numpy as jnp
from jax import lax
from jax.experimental import pallas as pl
from jax.experimental.pallas import tpu as pltpu
from jax.experimental.pallas import tpu_sc as plsc

B = 16384
D0 = 32
D1 = 64
DC = D0 + D1
H0 = D0 // 2
H1 = D1 // 2
NC = 2
NS = 16
NW = NC * NS
CHUNK = 128
CPW = B // (NW * CHUNK)
NW_ROWS = CPW * CHUNK

_mesh = plsc.VectorSubcoreMesh(core_axis_name="c", subcore_axis_name="s")


@functools.partial(
    pl.kernel,
    mesh=_mesh,
    out_type=jax.ShapeDtypeStruct((B, DC), jnp.float32),
    scratch_types=[
        pltpu.VMEM((NW_ROWS,), jnp.int32),
        pltpu.VMEM((NW_ROWS, H0), jnp.float32),
        pltpu.VMEM((NW_ROWS, H0), jnp.float32),
        pltpu.VMEM((NW_ROWS, H1), jnp.float32),
        pltpu.VMEM((NW_ROWS, H1), jnp.float32),
        pltpu.SemaphoreType.DMA,
        pltpu.SemaphoreType.DMA,
        pltpu.SemaphoreType.DMA,
    ],
    compiler_params=pltpu.CompilerParams(use_tc_tiling_on_sc=False),
)
def _pair_gather(t0a, t0b, t1a, t1b, ids, out,
                 idx_v, v0a, v0b, v1a, v1b, s_0, s_1, s_w):
    wid = lax.axis_index("s") * NC + lax.axis_index("c")
    base = wid * NW_ROWS
    pltpu.sync_copy(ids.at[pl.ds(base, NW_ROWS)], idx_v)
    gathers = []
    for c in range(CPW):
        isl = pl.ds(c * CHUNK, CHUNK)
        rows = pl.ds(c * CHUNK, CHUNK)
        gathers.append((
            pltpu.async_copy(t0a.at[idx_v.at[isl]], v0a.at[rows], s_0),
            pltpu.async_copy(t0b.at[idx_v.at[isl]], v0b.at[rows], s_0),
            pltpu.async_copy(t1a.at[idx_v.at[isl]], v1a.at[rows], s_1),
            pltpu.async_copy(t1b.at[idx_v.at[isl]], v1b.at[rows], s_1),
        ))
    orows = pl.ds(base, NW_ROWS)
    for g in gathers:
        g[0].wait()
        g[1].wait()
    w0 = pltpu.async_copy(v0a, out.at[orows, pl.ds(0, H0)], s_w)
    w1 = pltpu.async_copy(v0b, out.at[orows, pl.ds(H0, H0)], s_w)
    for g in gathers:
        g[2].wait()
        g[3].wait()
    w2 = pltpu.async_copy(v1a, out.at[orows, pl.ds(D0, H1)], s_w)
    w3 = pltpu.async_copy(v1b, out.at[orows, pl.ds(D0 + H1, H1)], s_w)
    w0.wait()
    w1.wait()
    w2.wait()
    w3.wait()


def kernel(item_table0, user_table0, item_table1, user_table1, item_ids, user_ids):
    item_out = _pair_gather(
        item_table0[:, :H0], item_table0[:, H0:],
        item_table1[:, :H1], item_table1[:, H1:], item_ids)
    user_out = _pair_gather(
        user_table0[:, :H0], user_table0[:, H0:],
        user_table1[:, :H1], user_table1[:, H1:], user_ids)
    return item_out, user_out


# speedup vs baseline: 2.4440x; 2.4440x over previous
import functools

import jax
import jax.numpy as jnp
from jax import lax
from jax.experimental import pallas as pl
from jax.experimental.pallas import tpu as pltpu
from jax.experimental.pallas import tpu_sc as plsc

B = 16384
D0 = 32
D1 = 64
DC = D0 + D1
NC = 2
NS = 16
NW = NC * NS
CHUNK = 128
CPW = B // (NW * CHUNK)
NW_ROWS = CPW * CHUNK

_mesh = plsc.VectorSubcoreMesh(core_axis_name="c", subcore_axis_name="s")


@functools.partial(
    pl.kernel,
    mesh=_mesh,
    out_type=jax.ShapeDtypeStruct((B, DC), jnp.float32),
    scratch_types=[
        pltpu.VMEM((NW_ROWS,), jnp.int32),
        pltpu.VMEM((NW_ROWS, D0), jnp.float32),
        pltpu.VMEM((NW_ROWS, D1), jnp.float32),
        pltpu.SemaphoreType.DMA,
        pltpu.SemaphoreType.DMA,
        pltpu.SemaphoreType.DMA,
    ],
    compiler_params=pltpu.CompilerParams(use_tc_tiling_on_sc=False),
)
def _pair_gather(t0, t1, ids, out, idx_v, v0, v1, s_0, s_1, s_w):
    wid = lax.axis_index("s") * NC + lax.axis_index("c")
    base = wid * NW_ROWS
    pltpu.sync_copy(ids.at[pl.ds(base, NW_ROWS)], idx_v)
    gathers = []
    for c in range(CPW):
        isl = pl.ds(c * CHUNK, CHUNK)
        rows = pl.ds(c * CHUNK, CHUNK)
        gathers.append((
            pltpu.async_copy(t0.at[idx_v.at[isl]], v0.at[rows], s_0),
            pltpu.async_copy(t1.at[idx_v.at[isl]], v1.at[rows], s_1),
        ))
    for c in range(CPW):
        gathers[c][0].wait()
    w0 = pltpu.async_copy(v0, out.at[pl.ds(base, NW_ROWS), pl.ds(0, D0)], s_w)
    for c in range(CPW):
        gathers[c][1].wait()
    w1 = pltpu.async_copy(v1, out.at[pl.ds(base, NW_ROWS), pl.ds(D0, D1)], s_w)
    w0.wait()
    w1.wait()


def kernel(item_table0, user_table0, item_table1, user_table1, item_ids, user_ids):
    item_out = _pair_gather(item_table0, item_table1, item_ids)
    user_out = _pair_gather(user_table0, user_table1, user_ids)
    return item_out, user_out
